# baseline (speedup 1.0000x reference)
import jax
import jax.numpy as jnp
from jax import lax
from jax.experimental import pallas as pl
from jax.experimental.pallas import tpu as pltpu
from jax.experimental.pallas import tpu_sc as plsc

_NC = 2
_NS = 16
_NW = _NC * _NS
_CHUNK = 128
_NBUF = 7


def _gather_body(T, xT_hbm, tab_hbm, out_hbm, idx_v, *rest):
    bufs = rest[:_NBUF]
    gsems = rest[_NBUF:2 * _NBUF]
    wsems = rest[2 * _NBUF:3 * _NBUF]
    wid = lax.axis_index("s") * _NC + lax.axis_index("c")
    coff = pl.multiple_of(wid * _CHUNK, 8)
    pltpu.sync_copy(xT_hbm.at[:, pl.ds(coff, _CHUNK)], idx_v)

    def start_gather(t, b):
        pltpu.async_copy(tab_hbm.at[idx_v.at[t]], bufs[b], gsems[b])

    def wait_gather(b):
        pltpu.make_async_copy(tab_hbm.at[idx_v.at[0]], bufs[b],
                              gsems[b]).wait()

    def start_write(t, b):
        pltpu.async_copy(bufs[b], out_hbm.at[t, pl.ds(coff, _CHUNK)],
                         wsems[b])

    def wait_write(b):
        pltpu.make_async_copy(bufs[b], out_hbm.at[0, pl.ds(0, _CHUNK)],
                              wsems[b]).wait()

    for b in range(_NBUF):
        start_gather(b, b)

    def outer(r, carry):
        for b in range(_NBUF):
            t = r * _NBUF + b

            @pl.when(t < T)
            def _():
                wait_gather(b)
                start_write(t, b)

        for b in range(_NBUF):
            tn = (r + 1) * _NBUF + b

            @pl.when(tn < T)
            def _():
                wait_write(b)
                start_gather(tn, b)

        return carry

    lax.fori_loop(0, -(-T // _NBUF), outer, 0)

    for b in range(_NBUF):
        wait_write(b)


def kernel(x, weight):
    B, T = x.shape
    V, D = weight.shape
    assert B % (_NW * _CHUNK) == 0 and T >= _NBUF

    xT = x.T.astype(jnp.int32)
    mesh = plsc.VectorSubcoreMesh(core_axis_name="c", subcore_axis_name="s")

    body = lambda *refs: _gather_body(T, *refs)
    out = pl.kernel(
        body,
        out_type=jax.ShapeDtypeStruct((T, B, D), jnp.float32),
        mesh=mesh,
        scratch_types=(
            [pltpu.VMEM((T, _CHUNK), jnp.int32)]
            + [pltpu.VMEM((_CHUNK, D), jnp.float32) for _ in range(_NBUF)]
            + [pltpu.SemaphoreType.DMA for _ in range(2 * _NBUF)]
        ),
    )(xT, weight)
    return out.transpose(1, 0, 2)
